# initial kernel scaffold (unmeasured)
import jax
import jax.numpy as jnp
from jax import lax
from jax.experimental import pallas as pl
from jax.experimental.pallas import tpu as pltpu


def kernel(
    x,
):
    def body(*refs):
        pass

    out_shape = jax.ShapeDtypeStruct(..., jnp.float32)
    return pl.pallas_call(body, out_shape=out_shape)(...)


# baseline (device time: 168094 ns/iter reference)
import jax
import jax.numpy as jnp
from jax import lax
from jax.experimental import pallas as pl
from jax.experimental.pallas import tpu as pltpu

P = 8


def kernel(x):
    m, n_global = x.shape
    n = n_global // P
    m_out = P * m

    x_bf = x.astype(jnp.bfloat16)

    def body(x_ref, out_ref, send_sems, recv_sems, local_sem):
        my = lax.axis_index("i")

        barrier = pltpu.get_barrier_semaphore()
        for d in range(1, P):
            peer = lax.rem(my + d, P)
            pl.semaphore_signal(
                barrier, inc=1,
                device_id=(peer,), device_id_type=pl.DeviceIdType.MESH,
            )
        pl.semaphore_wait(barrier, P - 1)

        local = pltpu.make_async_copy(
            x_ref.at[:, pl.ds(my * n, n)],
            out_ref.at[pl.ds(my * m, m), :],
            local_sem,
        )
        local.start()

        rdmas = []
        for d in range(1, P):
            dst = lax.rem(my + d, P)
            rdma = pltpu.make_async_remote_copy(
                src_ref=x_ref.at[:, pl.ds(dst * n, n)],
                dst_ref=out_ref.at[pl.ds(my * m, m), :],
                send_sem=send_sems.at[d],
                recv_sem=recv_sems.at[d],
                device_id=(dst,),
                device_id_type=pl.DeviceIdType.MESH,
            )
            rdma.start()
            rdmas.append(rdma)

        local.wait()
        for rdma in rdmas:
            rdma.wait()

    return pl.pallas_call(
        body,
        out_shape=jax.ShapeDtypeStruct((m_out, n), jnp.bfloat16),
        in_specs=[pl.BlockSpec(memory_space=pltpu.VMEM)],
        out_specs=pl.BlockSpec(memory_space=pltpu.VMEM),
        scratch_shapes=[
            pltpu.SemaphoreType.DMA((P,)),
            pltpu.SemaphoreType.DMA((P,)),
            pltpu.SemaphoreType.DMA,
        ],
        compiler_params=pltpu.CompilerParams(collective_id=0),
    )(x_bf)


# device time: 98770 ns/iter; 1.7019x vs baseline; 1.7019x over previous
import jax
import jax.numpy as jnp
from jax import lax
from jax.experimental import pallas as pl
from jax.experimental.pallas import tpu as pltpu

P = 8


def kernel(x):
    m, n_global = x.shape
    n = n_global // P
    m_out = P * m

    def body(x_ref, out_ref, xbf_ref, send_sems, recv_sems, local_sem):
        my = lax.axis_index("i")

        barrier = pltpu.get_barrier_semaphore()
        for d in range(1, P):
            peer = lax.rem(my + d, P)
            pl.semaphore_signal(
                barrier, inc=1,
                device_id=(peer,), device_id_type=pl.DeviceIdType.MESH,
            )

        xbf_ref[...] = x_ref[...].astype(jnp.bfloat16)

        pl.semaphore_wait(barrier, P - 1)

        local = pltpu.make_async_copy(
            xbf_ref.at[:, pl.ds(my * n, n)],
            out_ref.at[pl.ds(my * m, m), :],
            local_sem,
        )
        local.start()

        rdmas = []
        for d in range(1, P):
            dst = lax.rem(my + d, P)
            rdma = pltpu.make_async_remote_copy(
                src_ref=xbf_ref.at[:, pl.ds(dst * n, n)],
                dst_ref=out_ref.at[pl.ds(my * m, m), :],
                send_sem=send_sems.at[d],
                recv_sem=recv_sems.at[d],
                device_id=(dst,),
                device_id_type=pl.DeviceIdType.MESH,
            )
            rdma.start()
            rdmas.append(rdma)

        local.wait()
        for rdma in rdmas:
            rdma.wait()

    return pl.pallas_call(
        body,
        out_shape=jax.ShapeDtypeStruct((m_out, n), jnp.bfloat16),
        in_specs=[pl.BlockSpec(memory_space=pltpu.VMEM)],
        out_specs=pl.BlockSpec(memory_space=pltpu.MemorySpace.HBM),
        scratch_shapes=[
            pltpu.VMEM((m, n_global), jnp.bfloat16),
            pltpu.SemaphoreType.DMA((P,)),
            pltpu.SemaphoreType.DMA((P,)),
            pltpu.SemaphoreType.DMA,
        ],
        compiler_params=pltpu.CompilerParams(
            collective_id=0, vmem_limit_bytes=64 * 1024 * 1024
        ),
    )(x)
